# baseline (speedup 1.0000x reference)
import functools
import jax
import jax.numpy as jnp
import numpy as np
from jax import lax
from jax.experimental import pallas as pl
from jax.experimental.pallas import tpu as pltpu
from jax.experimental.pallas import tpu_sc as plsc

B = 16384
NC, NS = 2, 16
NW = NC * NS
CHUNK = B // NW

SRC = (0, 0, 1, 1, 2, 2)
DST = (1, 2, 0, 2, 0, 1)
IN_EDGES = ((2, 1, 4, 2), (0, 0, 5, 2), (1, 0, 3, 1))

_F32 = jnp.float32


def _rsqrt16(x):
    xi = lax.bitcast_convert_type(x, jnp.int32)
    yi = jnp.int32(0x5F3759DF) - lax.shift_right_arithmetic(xi, jnp.int32(1))
    y = lax.bitcast_convert_type(yi, _F32)
    for _ in range(3):
        y = y * (1.5 - 0.5 * x * y * y)
    return y


_GDN = lax.GatherDimensionNumbers(
    offset_dims=(), collapsed_slice_dims=(0,), start_index_map=(0,))


def _lane(v, k):
    idx = jnp.full((16, 1), k, jnp.int32)
    return lax.gather(v, idx, _GDN, (1,),
                      mode=lax.GatherScatterMode.PROMISE_IN_BOUNDS)


def _shuffle(v, idx):
    return lax.gather(v, idx, _GDN, (1,),
                      mode=lax.GatherScatterMode.PROMISE_IN_BOUNDS)


def _lane_sum(v):
    iota = lax.broadcasted_iota(jnp.int32, (16,), 0)
    for d in (1, 2, 4, 8):
        v = v + _shuffle(v, (iota ^ d).reshape(16, 1))
    return v


def _sc_gat_body(feat_hbm, wpack_hbm, pool_hbm, featv, wv, poolv):
    cid = lax.axis_index("c")
    sid = lax.axis_index("s")
    base = (sid * NC + cid) * CHUNK
    pltpu.sync_copy(feat_hbm.at[pl.ds(base, CHUNK)], featv)
    pltpu.sync_copy(wpack_hbm, wv)

    wl = [wv[k, :] for k in range(14)]
    wr = [wv[14 + k, :] for k in range(14)]
    we = [wv[28 + k, :] for k in range(3)]
    attv = wv[31, :]
    blv = wv[32, :]
    brv = wv[33, :]
    biasv = wv[34, :]

    def body(i, carry):
        fvec = [featv[i, pl.ds(16 * j, 16)] for j in range(3)]
        f = [[_lane(fvec[j], k) for k in range(14)] for j in range(3)]

        xl, xr = [], []
        for j in range(3):
            al, ar = blv, brv
            for k in range(14):
                al = al + f[j][k] * wl[k]
                ar = ar + f[j][k] * wr[k]
            xl.append(al)
            xr.append(ar)

        evec = {}
        for (s, d) in ((0, 1), (0, 2), (1, 2)):
            cx = f[d][0] - f[s][0]
            cy = f[d][1] - f[s][1]
            d2v = cx * cx + cy * cy
            distv = d2v * _rsqrt16(d2v)
            t = distv * we[2]
            u = cx * we[0] + cy * we[1]
            evec[(s, d)] = t + u
            evec[(d, s)] = t - u

        alphas = []
        for e in range(6):
            s, d = SRC[e], DST[e]
            m = xl[s] + xr[d] + evec[(s, d)]
            m = jnp.where(m > 0, m, 0.2 * m)
            alphas.append(_lane_sum(m * attv))

        pool = jnp.zeros((16,), _F32)
        for dd in range(3):
            e1, s1, e2, s2 = IN_EDGES[dd]
            a1, a2 = alphas[e1], alphas[e2]
            amax = jnp.maximum(a1, a2)
            x1 = jnp.exp(a1 - amax)
            x2 = jnp.exp(a2 - amax)
            den = x1 + x2 + 1e-16
            o = (x1 / den) * xl[s1] + (x2 / den) * xl[s2] + biasv
            pool = pool + jnp.maximum(o, 0.0)
        poolv[i, :] = pool
        return carry

    lax.fori_loop(0, CHUNK, body, 0)
    pltpu.sync_copy(poolv, pool_hbm.at[pl.ds(base, CHUNK)])


def _sc_gat(feat, wpack):
    mesh = plsc.VectorSubcoreMesh(core_axis_name="c", subcore_axis_name="s")
    return pl.kernel(
        _sc_gat_body,
        mesh=mesh,
        compiler_params=pltpu.CompilerParams(use_tc_tiling_on_sc=False),
        out_type=jax.ShapeDtypeStruct((B, 16), _F32),
        scratch_types=[
            pltpu.VMEM((CHUNK, 48), _F32),
            pltpu.VMEM((35, 16), _F32),
            pltpu.VMEM((CHUNK, 16), _F32),
        ],
    )(feat, wpack)


TILE_H = 2048


def _dot(a, b):
    return jax.lax.dot(a, b, preferred_element_type=_F32)


def _tc_head_body(pool_ref, lm_ref, Wl_ref, bl_ref, bias_ref, W1_ref, b1_ref,
                  W2_ref, b2_ref, out_ref):
    Wl = Wl_ref[:, :]
    V = Wl[6:10, :] + Wl[10:14, :]
    A = jnp.concatenate([
        Wl[0:1, :] - V[0:1, :] - V[2:3, :],
        Wl[1:2, :] - V[1:2, :] - V[3:4, :],
        V,
    ], axis=0)
    xlobj = _dot(lm_ref[:, :], A) + bl_ref[:, :] + bias_ref[:, :]
    objpool = 3.0 * jnp.maximum(xlobj, 0.0)
    h = jnp.concatenate([pool_ref[:, :], objpool], axis=1)
    hid = jnp.maximum(_dot(h, W1_ref[:, :]) + b1_ref[:, :], 0.0)
    out_ref[:, :] = _dot(hid, W2_ref[:, :]) + b2_ref[:, :]


def _tc_head(pool, lm, W_l, b_l, bias, W1, b1, W2, b2):
    grid = (B // TILE_H,)
    full = lambda shape: pl.BlockSpec(shape, lambda i: (0, 0))
    return pl.pallas_call(
        _tc_head_body,
        grid=grid,
        in_specs=[
            pl.BlockSpec((TILE_H, 16), lambda i: (i, 0)),
            pl.BlockSpec((TILE_H, 6), lambda i: (i, 0)),
            full((14, 16)),
            full((1, 16)),
            full((1, 16)),
            full((32, 128)),
            full((1, 128)),
            full((128, 32)),
            full((1, 32)),
        ],
        out_specs=pl.BlockSpec((TILE_H, 32), lambda i: (i, 0)),
        out_shape=jax.ShapeDtypeStruct((B, 32), _F32),
    )(pool, lm, W_l, b_l, bias, W1, b1, W2, b2)


@jax.jit
def _run(feat, lm, wpack, W_l, b_l, bias, W1, b1, W2, b2):
    pool = _sc_gat(feat, wpack)
    return _tc_head(pool, lm, W_l, b_l, bias, W1, b1, W2, b2)


def kernel(agent_pos, landmark_pos, agent_vel, other_pos, relative_landmark_pos,
           W_l, b_l, W_r, b_r, W_e, att, bias, W1, b1, W2, b2):
    b = agent_pos.shape[0]
    feat = jnp.concatenate(
        [agent_pos, agent_vel, relative_landmark_pos, other_pos,
         jnp.zeros((b, 3, 2), _F32)], axis=2).reshape(b, 48)
    lm = landmark_pos.reshape(b, 6)
    wpack = jnp.concatenate(
        [W_l, W_r, W_e, att[None, :], b_l[None, :], b_r[None, :],
         bias[None, :]], axis=0)
    return _run(feat, lm, wpack, W_l, b_l[None, :], bias[None, :],
                W1, b1[None, :], W2, b2[None, :])


# speedup vs baseline: 1.0108x; 1.0108x over previous
import functools
import jax
import jax.numpy as jnp
import numpy as np
from jax import lax
from jax.experimental import pallas as pl
from jax.experimental.pallas import tpu as pltpu
from jax.experimental.pallas import tpu_sc as plsc

B = 16384
NC, NS = 2, 16
NW = NC * NS
CHUNK = B // NW

SRC = (0, 0, 1, 1, 2, 2)
DST = (1, 2, 0, 2, 0, 1)
IN_EDGES = ((2, 1, 4, 2), (0, 0, 5, 2), (1, 0, 3, 1))

_F32 = jnp.float32


def _rsqrt16(x):
    xi = lax.bitcast_convert_type(x, jnp.int32)
    yi = jnp.int32(0x5F3759DF) - lax.shift_right_arithmetic(xi, jnp.int32(1))
    y = lax.bitcast_convert_type(yi, _F32)
    for _ in range(3):
        y = y * (1.5 - 0.5 * x * y * y)
    return y


_GDN = lax.GatherDimensionNumbers(
    offset_dims=(), collapsed_slice_dims=(0,), start_index_map=(0,))


def _lane(v, k):
    idx = jnp.full((16, 1), k, jnp.int32)
    return lax.gather(v, idx, _GDN, (1,),
                      mode=lax.GatherScatterMode.PROMISE_IN_BOUNDS)


def _shuffle(v, idx):
    return lax.gather(v, idx, _GDN, (1,),
                      mode=lax.GatherScatterMode.PROMISE_IN_BOUNDS)


def _lane_sum(v):
    iota = lax.broadcasted_iota(jnp.int32, (16,), 0)
    for d in (1, 2, 4, 8):
        v = v + _shuffle(v, (iota ^ d).reshape(16, 1))
    return v


def _sc_gat_body(feat_hbm, wpack_hbm, pool_hbm, featv, wv, poolv):
    cid = lax.axis_index("c")
    sid = lax.axis_index("s")
    base = (sid * NC + cid) * CHUNK
    pltpu.sync_copy(feat_hbm.at[pl.ds(base, CHUNK)], featv)
    pltpu.sync_copy(wpack_hbm, wv)

    wl = [wv[k, :] for k in range(14)]
    wr = [wv[14 + k, :] for k in range(14)]
    we = [wv[28 + k, :] for k in range(3)]
    attv = wv[31, :]
    blv = wv[32, :]
    brv = wv[33, :]
    biasv = wv[34, :]

    @plsc.parallel_loop(0, CHUNK, step=1, unroll=4)
    def body(i):
        fvec = [featv[i, pl.ds(16 * j, 16)] for j in range(3)]
        f = [[_lane(fvec[j], k) for k in range(14)] for j in range(3)]

        xl, xr = [], []
        for j in range(3):
            al, ar = blv, brv
            for k in range(14):
                al = al + f[j][k] * wl[k]
                ar = ar + f[j][k] * wr[k]
            xl.append(al)
            xr.append(ar)

        evec = {}
        for (s, d) in ((0, 1), (0, 2), (1, 2)):
            cx = f[d][0] - f[s][0]
            cy = f[d][1] - f[s][1]
            d2v = cx * cx + cy * cy
            distv = d2v * _rsqrt16(d2v)
            t = distv * we[2]
            u = cx * we[0] + cy * we[1]
            evec[(s, d)] = t + u
            evec[(d, s)] = t - u

        alphas = []
        for e in range(6):
            s, d = SRC[e], DST[e]
            m = xl[s] + xr[d] + evec[(s, d)]
            m = jnp.where(m > 0, m, 0.2 * m)
            alphas.append(_lane_sum(m * attv))

        pool = jnp.zeros((16,), _F32)
        for dd in range(3):
            e1, s1, e2, s2 = IN_EDGES[dd]
            a1, a2 = alphas[e1], alphas[e2]
            amax = jnp.maximum(a1, a2)
            x1 = jnp.exp(a1 - amax)
            x2 = jnp.exp(a2 - amax)
            den = x1 + x2 + 1e-16
            o = (x1 / den) * xl[s1] + (x2 / den) * xl[s2] + biasv
            pool = pool + jnp.maximum(o, 0.0)
        poolv[i, :] = pool

    pltpu.sync_copy(poolv, pool_hbm.at[pl.ds(base, CHUNK)])


def _sc_gat(feat, wpack):
    mesh = plsc.VectorSubcoreMesh(core_axis_name="c", subcore_axis_name="s")
    return pl.kernel(
        _sc_gat_body,
        mesh=mesh,
        compiler_params=pltpu.CompilerParams(use_tc_tiling_on_sc=False),
        out_type=jax.ShapeDtypeStruct((B, 16), _F32),
        scratch_types=[
            pltpu.VMEM((CHUNK, 48), _F32),
            pltpu.VMEM((35, 16), _F32),
            pltpu.VMEM((CHUNK, 16), _F32),
        ],
    )(feat, wpack)


TILE_H = 2048


def _dot(a, b):
    return jax.lax.dot(a, b, preferred_element_type=_F32)


def _tc_head_body(pool_ref, lm_ref, Wl_ref, bl_ref, bias_ref, W1_ref, b1_ref,
                  W2_ref, b2_ref, out_ref):
    Wl = Wl_ref[:, :]
    V = Wl[6:10, :] + Wl[10:14, :]
    A = jnp.concatenate([
        Wl[0:1, :] - V[0:1, :] - V[2:3, :],
        Wl[1:2, :] - V[1:2, :] - V[3:4, :],
        V,
    ], axis=0)
    xlobj = _dot(lm_ref[:, :], A) + bl_ref[:, :] + bias_ref[:, :]
    objpool = 3.0 * jnp.maximum(xlobj, 0.0)
    h = jnp.concatenate([pool_ref[:, :], objpool], axis=1)
    hid = jnp.maximum(_dot(h, W1_ref[:, :]) + b1_ref[:, :], 0.0)
    out_ref[:, :] = _dot(hid, W2_ref[:, :]) + b2_ref[:, :]


def _tc_head(pool, lm, W_l, b_l, bias, W1, b1, W2, b2):
    grid = (B // TILE_H,)
    full = lambda shape: pl.BlockSpec(shape, lambda i: (0, 0))
    return pl.pallas_call(
        _tc_head_body,
        grid=grid,
        in_specs=[
            pl.BlockSpec((TILE_H, 16), lambda i: (i, 0)),
            pl.BlockSpec((TILE_H, 6), lambda i: (i, 0)),
            full((14, 16)),
            full((1, 16)),
            full((1, 16)),
            full((32, 128)),
            full((1, 128)),
            full((128, 32)),
            full((1, 32)),
        ],
        out_specs=pl.BlockSpec((TILE_H, 32), lambda i: (i, 0)),
        out_shape=jax.ShapeDtypeStruct((B, 32), _F32),
    )(pool, lm, W_l, b_l, bias, W1, b1, W2, b2)


@jax.jit
def _run(feat, lm, wpack, W_l, b_l, bias, W1, b1, W2, b2):
    pool = _sc_gat(feat, wpack)
    return _tc_head(pool, lm, W_l, b_l, bias, W1, b1, W2, b2)


def kernel(agent_pos, landmark_pos, agent_vel, other_pos, relative_landmark_pos,
           W_l, b_l, W_r, b_r, W_e, att, bias, W1, b1, W2, b2):
    b = agent_pos.shape[0]
    feat = jnp.concatenate(
        [agent_pos, agent_vel, relative_landmark_pos, other_pos,
         jnp.zeros((b, 3, 2), _F32)], axis=2).reshape(b, 48)
    lm = landmark_pos.reshape(b, 6)
    wpack = jnp.concatenate(
        [W_l, W_r, W_e, att[None, :], b_l[None, :], b_r[None, :],
         bias[None, :]], axis=0)
    return _run(feat, lm, wpack, W_l, b_l[None, :], bias[None, :],
                W1, b1[None, :], W2, b2[None, :])
